# baseline (speedup 1.0000x reference)
import functools

import numpy as np

import jax
import jax.numpy as jnp
from jax import lax
from jax.experimental import pallas as pl
from jax.experimental.pallas import tpu as pltpu
from jax.experimental.pallas import tpu_sc as plsc

_BATCH = 16384
_HIST = 50
_VOCAB = 1_000_000
_DIM = 32

_NW = 32
_RPW = _BATCH // _NW
_IPW = _RPW * _HIST
_IDX_ROWS = _IPW // 128

_TC_BLK = 32768
_TC_GRID = pl.cdiv(_VOCAB, _TC_BLK)
_VOCAB_PAD = _TC_GRID * _TC_BLK


def _matvec_body(emb_ref, wt_ref, b_ref, t_ref):
    acc = jax.lax.dot_general(
        wt_ref[...], emb_ref[...], (((1,), (1,)), ((), ())),
        preferred_element_type=jnp.float32)
    t_ref[...] = (acc + b_ref[...]) * (1.0 / _HIST)


def _build_table(emb_table, Wt, b2):
    return pl.pallas_call(
        _matvec_body,
        grid=(_TC_GRID,),
        in_specs=[
            pl.BlockSpec((_TC_BLK, _DIM), lambda i: (i, 0)),
            pl.BlockSpec((1, _DIM), lambda i: (0, 0)),
            pl.BlockSpec((1, 1), lambda i: (0, 0)),
        ],
        out_specs=pl.BlockSpec((1, _TC_BLK), lambda i: (0, i)),
        out_shape=jax.ShapeDtypeStruct((1, _VOCAB_PAD), jnp.float32),
    )(emb_table, Wt, b2)


_PERM_NP = (
    _IPW * np.arange(_NW, dtype=np.int64)[:, None]
    + (_HIST * np.arange(_RPW, dtype=np.int64)[None, :]
       + np.arange(_HIST, dtype=np.int64)[:, None]).reshape(-1)[None, :]
).astype(np.int32)


def _sc_pool_body(xf_hbm, perm_hbm, t_hbm, out_hbm, pidx_v, idx_v, vals_v,
                  out_v, sem):
    wid = lax.axis_index("s") * 2 + lax.axis_index("c")
    pltpu.sync_copy(perm_hbm.at[wid], pidx_v)
    pltpu.async_copy(xf_hbm.at[pidx_v], idx_v, sem).wait()
    pltpu.async_copy(t_hbm.at[idx_v], vals_v, sem).wait()
    for g in range(_RPW // 16):
        acc = vals_v[pl.ds(16 * g, 16)]
        for l in range(1, _HIST):
            acc = acc + vals_v[pl.ds(512 * l + 16 * g, 16)]
        out_v[pl.ds(16 * g, 16)] = 1.0 / (1.0 + jnp.exp(-acc))
    pltpu.sync_copy(out_v, out_hbm.at[pl.ds(wid * _RPW, _RPW)])


_sc_pool = functools.partial(
    pl.kernel,
    out_type=jax.ShapeDtypeStruct((_BATCH,), jnp.float32),
    mesh=plsc.VectorSubcoreMesh(core_axis_name="c", subcore_axis_name="s"),
    scratch_types=[
        pltpu.VMEM((_IPW,), jnp.int32),
        pltpu.VMEM((_IPW,), jnp.int32),
        pltpu.VMEM((_IPW,), jnp.float32),
        pltpu.VMEM((_RPW,), jnp.float32),
        pltpu.SemaphoreType.DMA,
    ],
)(_sc_pool_body)


def kernel(x, emb_table, W, b):
    t2 = _build_table(emb_table, W.reshape(1, _DIM), b.reshape(1, 1))
    y = _sc_pool(x.reshape(-1), jnp.asarray(_PERM_NP), t2.reshape(-1))
    return y.reshape(_BATCH, 1)


# speedup vs baseline: 5.1414x; 5.1414x over previous
import functools

import jax
import jax.numpy as jnp
from jax import lax
from jax.experimental import pallas as pl
from jax.experimental.pallas import tpu as pltpu
from jax.experimental.pallas import tpu_sc as plsc

_BATCH = 16384
_HIST = 50
_VOCAB = 1_000_000
_DIM = 32

_NW = 32
_RPW = _BATCH // _NW
_IPW = _RPW * _HIST
_IDX_ROWS = _IPW // 128

_TC_BLK = 32768
_TC_GRID = pl.cdiv(_VOCAB, _TC_BLK)
_VOCAB_PAD = _TC_GRID * _TC_BLK


def _matvec_body(embt_ref, wt_ref, b_ref, t_ref):
    acc = jnp.dot(wt_ref[...], embt_ref[...],
                  preferred_element_type=jnp.float32)
    t_ref[...] = (acc + b_ref[...]) * (1.0 / _HIST)


def _build_table(emb_t, Wt, b2):
    return pl.pallas_call(
        _matvec_body,
        grid=(_TC_GRID,),
        in_specs=[
            pl.BlockSpec((_DIM, _TC_BLK), lambda i: (0, i)),
            pl.BlockSpec((1, _DIM), lambda i: (0, 0)),
            pl.BlockSpec((1, 1), lambda i: (0, 0)),
        ],
        out_specs=pl.BlockSpec((1, _TC_BLK), lambda i: (0, i)),
        out_shape=jax.ShapeDtypeStruct((1, _VOCAB_PAD), jnp.float32),
    )(emb_t, Wt, b2)


def _sc_pool_body(xt_hbm, t_hbm, out_hbm, idx_v, vals_v, out_v, sem, isem):
    wid = lax.axis_index("s") * 2 + lax.axis_index("c")
    for l in range(_HIST):
        pltpu.async_copy(
            xt_hbm.at[pl.ds(l * _BATCH + wid * _RPW, _RPW)],
            idx_v.at[pl.ds(l * _RPW, _RPW)], isem)
    for l in range(_HIST):
        pltpu.make_async_copy(
            xt_hbm.at[pl.ds(l * _BATCH + wid * _RPW, _RPW)],
            idx_v.at[pl.ds(l * _RPW, _RPW)], isem).wait()
    pltpu.async_copy(t_hbm.at[idx_v], vals_v, sem).wait()
    for g in range(_RPW // 16):
        acc = vals_v[pl.ds(16 * g, 16)]
        for l in range(1, _HIST):
            acc = acc + vals_v[pl.ds(512 * l + 16 * g, 16)]
        out_v[pl.ds(16 * g, 16)] = 1.0 / (1.0 + jnp.exp(-acc))
    pltpu.sync_copy(out_v, out_hbm.at[pl.ds(wid * _RPW, _RPW)])


_sc_pool = functools.partial(
    pl.kernel,
    out_type=jax.ShapeDtypeStruct((_BATCH,), jnp.float32),
    mesh=plsc.VectorSubcoreMesh(core_axis_name="c", subcore_axis_name="s"),
    scratch_types=[
        pltpu.VMEM((_IPW,), jnp.int32),
        pltpu.VMEM((_IPW,), jnp.float32),
        pltpu.VMEM((_RPW,), jnp.float32),
        pltpu.SemaphoreType.DMA,
        pltpu.SemaphoreType.DMA,
    ],
)(_sc_pool_body)


def kernel(x, emb_table, W, b):
    t2 = _build_table(emb_table.T, W.reshape(1, _DIM), b.reshape(1, 1))
    y = _sc_pool(x.T.reshape(-1), t2.reshape(-1))
    return y.reshape(_BATCH, 1)


# speedup vs baseline: 5.4248x; 1.0551x over previous
import functools

import jax
import jax.numpy as jnp
from jax import lax
from jax.experimental import pallas as pl
from jax.experimental.pallas import tpu as pltpu
from jax.experimental.pallas import tpu_sc as plsc

_BATCH = 16384
_HIST = 50
_VOCAB = 1_000_000
_DIM = 32

_NW = 32
_RPW = _BATCH // _NW
_IPW = _RPW * _HIST
_IDX_ROWS = _IPW // 128

_TC_BLK = 65536
_TC_GRID = pl.cdiv(_VOCAB, _TC_BLK)
_VOCAB_PAD = _TC_GRID * _TC_BLK

_NCH = 5
_LPC = _HIST // _NCH
_CHW = _LPC * _RPW


def _matvec_body(embt_ref, wt_ref, b_ref, t_ref):
    acc = jnp.dot(wt_ref[...], embt_ref[...],
                  preferred_element_type=jnp.float32)
    t_ref[...] = (acc + b_ref[...]) * (1.0 / _HIST)


def _build_table(emb_t, Wt, b2):
    return pl.pallas_call(
        _matvec_body,
        grid=(_TC_GRID,),
        in_specs=[
            pl.BlockSpec((_DIM, _TC_BLK), lambda i: (0, i)),
            pl.BlockSpec((1, _DIM), lambda i: (0, 0)),
            pl.BlockSpec((1, 1), lambda i: (0, 0)),
        ],
        out_specs=pl.BlockSpec((1, _TC_BLK), lambda i: (0, i)),
        out_shape=jax.ShapeDtypeStruct((1, _VOCAB_PAD), jnp.float32),
    )(emb_t, Wt, b2)


def _sc_pool_body(xt_hbm, t_hbm, out_hbm, idx_v, vals_v, out_v, isems, gsems):
    wid = lax.axis_index("s") * 2 + lax.axis_index("c")

    def _idx_copy(l):
        return pltpu.make_async_copy(
            xt_hbm.at[pl.ds(l * _BATCH + wid * _RPW, _RPW)],
            idx_v.at[pl.ds(l * _RPW, _RPW)], isems.at[l // _LPC])

    def _gather_copy(c):
        return pltpu.make_async_copy(
            t_hbm.at[idx_v.at[pl.ds(_CHW * c, _CHW)]],
            vals_v.at[pl.ds(_CHW * c, _CHW)], gsems.at[c])

    for l in range(_HIST):
        _idx_copy(l).start()
    for c in range(_NCH):
        for l in range(_LPC * c, _LPC * (c + 1)):
            _idx_copy(l).wait()
        _gather_copy(c).start()
    for c in range(_NCH):
        _gather_copy(c).wait()
        for g in range(_RPW // 16):
            acc = vals_v[pl.ds(_CHW * c + 16 * g, 16)]
            for l in range(_LPC * c + 1, _LPC * (c + 1)):
                acc = acc + vals_v[pl.ds(512 * l + 16 * g, 16)]
            if c == 0:
                out_v[pl.ds(16 * g, 16)] = acc
            else:
                out_v[pl.ds(16 * g, 16)] = out_v[pl.ds(16 * g, 16)] + acc
    for g in range(_RPW // 16):
        z = out_v[pl.ds(16 * g, 16)]
        out_v[pl.ds(16 * g, 16)] = 1.0 / (1.0 + jnp.exp(-z))
    pltpu.sync_copy(out_v, out_hbm.at[pl.ds(wid * _RPW, _RPW)])


_sc_pool = functools.partial(
    pl.kernel,
    out_type=jax.ShapeDtypeStruct((_BATCH,), jnp.float32),
    mesh=plsc.VectorSubcoreMesh(core_axis_name="c", subcore_axis_name="s"),
    scratch_types=[
        pltpu.VMEM((_IPW,), jnp.int32),
        pltpu.VMEM((_IPW,), jnp.float32),
        pltpu.VMEM((_RPW,), jnp.float32),
        pltpu.SemaphoreType.DMA((_NCH,)),
        pltpu.SemaphoreType.DMA((_NCH,)),
    ],
)(_sc_pool_body)


def kernel(x, emb_table, W, b):
    t2 = _build_table(emb_table.T, W.reshape(1, _DIM), b.reshape(1, 1))
    y = _sc_pool(x.T.reshape(-1), t2.reshape(-1))
    return y.reshape(_BATCH, 1)


# speedup vs baseline: 6.7148x; 1.2378x over previous
import functools

import jax
import jax.numpy as jnp
from jax import lax
from jax.experimental import pallas as pl
from jax.experimental.pallas import tpu as pltpu
from jax.experimental.pallas import tpu_sc as plsc

_BATCH = 16384
_HIST = 50
_VOCAB = 1_000_000
_DIM = 32

_NW = 32
_RPW = _BATCH // _NW
_IPW = _RPW * _HIST
_IDX_ROWS = _IPW // 128

_TC_BLK = 65536
_TC_GRID = pl.cdiv(_VOCAB, _TC_BLK)
_VOCAB_PAD = _TC_GRID * _TC_BLK

_NCH = 5
_LPC = _HIST // _NCH
_CHW = _LPC * _RPW


def _matvec_body(embt_ref, wt_ref, b_ref, t_ref):
    acc = jnp.dot(wt_ref[...], embt_ref[...],
                  preferred_element_type=jnp.float32)
    t_ref[...] = (acc + b_ref[...]) * (1.0 / _HIST)


def _build_table(emb_t, Wt, b2):
    return pl.pallas_call(
        _matvec_body,
        grid=(_TC_GRID,),
        in_specs=[
            pl.BlockSpec((_DIM, _TC_BLK), lambda i: (0, i)),
            pl.BlockSpec((1, _DIM), lambda i: (0, 0)),
            pl.BlockSpec((1, 1), lambda i: (0, 0)),
        ],
        out_specs=pl.BlockSpec((1, _TC_BLK), lambda i: (0, i)),
        out_shape=jax.ShapeDtypeStruct((1, _VOCAB_PAD), jnp.float32),
    )(emb_t, Wt, b2)


_STG = _VOCAB_PAD // 16


def _sc_pool_body(xt_hbm, t_hbm, out_hbm, idx_v, vals_v, out_v, t_sh,
                  isems, gsems):
    sub = lax.axis_index("s")
    wid = sub * 2 + lax.axis_index("c")

    def _idx_copy(l):
        return pltpu.make_async_copy(
            xt_hbm.at[pl.ds(l * _BATCH + wid * _RPW, _RPW)],
            idx_v.at[pl.ds(l * _RPW, _RPW)], isems.at[l // _LPC])

    def _gather_copy(c):
        return pltpu.make_async_copy(
            t_sh.at[idx_v.at[pl.ds(_CHW * c, _CHW)]],
            vals_v.at[pl.ds(_CHW * c, _CHW)], gsems.at[c])

    for l in range(_HIST):
        _idx_copy(l).start()
    pltpu.sync_copy(t_hbm.at[pl.ds(sub * _STG, _STG)],
                    t_sh.at[pl.ds(sub * _STG, _STG)])
    plsc.subcore_barrier()
    for c in range(_NCH):
        for l in range(_LPC * c, _LPC * (c + 1)):
            _idx_copy(l).wait()
        _gather_copy(c).start()
    for c in range(_NCH):
        _gather_copy(c).wait()
        for g in range(_RPW // 16):
            acc = vals_v[pl.ds(_CHW * c + 16 * g, 16)]
            for l in range(_LPC * c + 1, _LPC * (c + 1)):
                acc = acc + vals_v[pl.ds(512 * l + 16 * g, 16)]
            if c == 0:
                out_v[pl.ds(16 * g, 16)] = acc
            else:
                out_v[pl.ds(16 * g, 16)] = out_v[pl.ds(16 * g, 16)] + acc
    for g in range(_RPW // 16):
        z = out_v[pl.ds(16 * g, 16)]
        out_v[pl.ds(16 * g, 16)] = 1.0 / (1.0 + jnp.exp(-z))
    pltpu.sync_copy(out_v, out_hbm.at[pl.ds(wid * _RPW, _RPW)])


_sc_pool = functools.partial(
    pl.kernel,
    out_type=jax.ShapeDtypeStruct((_BATCH,), jnp.float32),
    mesh=plsc.VectorSubcoreMesh(core_axis_name="c", subcore_axis_name="s"),
    scratch_types=[
        pltpu.VMEM((_IPW,), jnp.int32),
        pltpu.VMEM((_IPW,), jnp.float32),
        pltpu.VMEM((_RPW,), jnp.float32),
        pltpu.VMEM_SHARED((_VOCAB_PAD,), jnp.float32),
        pltpu.SemaphoreType.DMA((_NCH,)),
        pltpu.SemaphoreType.DMA((_NCH,)),
    ],
)(_sc_pool_body)


def kernel(x, emb_table, W, b):
    t2 = _build_table(emb_table.T, W.reshape(1, _DIM), b.reshape(1, 1))
    y = _sc_pool(x.T.reshape(-1), t2.reshape(-1))
    return y.reshape(_BATCH, 1)


# speedup vs baseline: 6.9948x; 1.0417x over previous
import functools

import jax
import jax.numpy as jnp
from jax import lax
from jax.experimental import pallas as pl
from jax.experimental.pallas import tpu as pltpu
from jax.experimental.pallas import tpu_sc as plsc

_BATCH = 16384
_HIST = 50
_VOCAB = 1_000_000
_DIM = 32

_NW = 32
_RPW = _BATCH // _NW
_IPW = _RPW * _HIST
_IDX_ROWS = _IPW // 128

_TC_BLK = 131072
_TC_GRID = pl.cdiv(_VOCAB, _TC_BLK)
_VOCAB_PAD = _TC_GRID * _TC_BLK

_NCH = 5
_LPC = _HIST // _NCH
_CHW = _LPC * _RPW


def _matvec_body(embt_ref, wt_ref, b_ref, t_ref):
    acc = jnp.dot(wt_ref[...], embt_ref[...],
                  preferred_element_type=jnp.float32)
    t_ref[...] = (acc + b_ref[...]) * (1.0 / _HIST)


def _build_table(emb_t, Wt, b2):
    return pl.pallas_call(
        _matvec_body,
        grid=(_TC_GRID,),
        in_specs=[
            pl.BlockSpec((_DIM, _TC_BLK), lambda i: (0, i)),
            pl.BlockSpec((1, _DIM), lambda i: (0, 0)),
            pl.BlockSpec((1, 1), lambda i: (0, 0)),
        ],
        out_specs=pl.BlockSpec((1, _TC_BLK), lambda i: (0, i)),
        out_shape=jax.ShapeDtypeStruct((1, _VOCAB_PAD), jnp.float32),
    )(emb_t, Wt, b2)


_STG = _VOCAB_PAD // 16


def _sc_pool_body(xt_hbm, t_hbm, out_hbm, idx_v, vals_v, out_v, t_sh,
                  isems, gsems):
    sub = lax.axis_index("s")
    wid = sub * 2 + lax.axis_index("c")

    def _idx_copy(l):
        return pltpu.make_async_copy(
            xt_hbm.at[l, pl.ds(wid * _RPW, _RPW)],
            idx_v.at[pl.ds(l * _RPW, _RPW)], isems.at[l // _LPC])

    def _gather_copy(c):
        return pltpu.make_async_copy(
            t_sh.at[idx_v.at[pl.ds(_CHW * c, _CHW)]],
            vals_v.at[pl.ds(_CHW * c, _CHW)], gsems.at[c])

    for l in range(_HIST):
        _idx_copy(l).start()
    pltpu.sync_copy(t_hbm.at[pl.ds(sub * _STG, _STG)],
                    t_sh.at[pl.ds(sub * _STG, _STG)])
    plsc.subcore_barrier()
    for c in range(_NCH):
        for l in range(_LPC * c, _LPC * (c + 1)):
            _idx_copy(l).wait()
        _gather_copy(c).start()
    for c in range(_NCH):
        _gather_copy(c).wait()
        for g in range(_RPW // 16):
            acc = vals_v[pl.ds(_CHW * c + 16 * g, 16)]
            for l in range(_LPC * c + 1, _LPC * (c + 1)):
                acc = acc + vals_v[pl.ds(512 * l + 16 * g, 16)]
            if c == 0:
                out_v[pl.ds(16 * g, 16)] = acc
            else:
                out_v[pl.ds(16 * g, 16)] = out_v[pl.ds(16 * g, 16)] + acc
    for g in range(_RPW // 16):
        z = out_v[pl.ds(16 * g, 16)]
        out_v[pl.ds(16 * g, 16)] = 1.0 / (1.0 + jnp.exp(-z))
    pltpu.sync_copy(out_v, out_hbm.at[pl.ds(wid * _RPW, _RPW)])


_sc_pool = functools.partial(
    pl.kernel,
    out_type=jax.ShapeDtypeStruct((_BATCH,), jnp.float32),
    mesh=plsc.VectorSubcoreMesh(core_axis_name="c", subcore_axis_name="s"),
    scratch_types=[
        pltpu.VMEM((_IPW,), jnp.int32),
        pltpu.VMEM((_IPW,), jnp.float32),
        pltpu.VMEM((_RPW,), jnp.float32),
        pltpu.VMEM_SHARED((_VOCAB_PAD,), jnp.float32),
        pltpu.SemaphoreType.DMA((_NCH,)),
        pltpu.SemaphoreType.DMA((_NCH,)),
    ],
)(_sc_pool_body)


def kernel(x, emb_table, W, b):
    t2 = _build_table(emb_table.T, W.reshape(1, _DIM), b.reshape(1, 1))
    y = _sc_pool(x.T, t2.reshape(-1))
    return y.reshape(_BATCH, 1)


# speedup vs baseline: 7.1957x; 1.0287x over previous
import functools

import jax
import jax.numpy as jnp
from jax import lax
from jax.experimental import pallas as pl
from jax.experimental.pallas import tpu as pltpu
from jax.experimental.pallas import tpu_sc as plsc

_BATCH = 16384
_HIST = 50
_VOCAB = 1_000_000
_DIM = 32

_NW = 32
_RPW = _BATCH // _NW
_IPW = _RPW * _HIST
_IDX_ROWS = _IPW // 128

_TC_BLK = 65536
_TC_GRID = pl.cdiv(_VOCAB, _TC_BLK)
_VOCAB_PAD = _TC_GRID * _TC_BLK

_NCH = 5
_LPC = _HIST // _NCH
_CHW = _LPC * _RPW


def _matvec_body(embt_ref, wt_ref, b_ref, t_ref):
    acc = jnp.dot(wt_ref[...], embt_ref[...],
                  preferred_element_type=jnp.float32)
    t_ref[...] = (acc + b_ref[...]) * (1.0 / _HIST)


def _build_table(emb_t, Wt, b2):
    return pl.pallas_call(
        _matvec_body,
        grid=(_TC_GRID,),
        in_specs=[
            pl.BlockSpec((_DIM, _TC_BLK), lambda i: (0, i)),
            pl.BlockSpec((1, _DIM), lambda i: (0, 0)),
            pl.BlockSpec((1, 1), lambda i: (0, 0)),
        ],
        out_specs=pl.BlockSpec((1, _TC_BLK), lambda i: (0, i)),
        out_shape=jax.ShapeDtypeStruct((1, _VOCAB_PAD), jnp.float32),
    )(emb_t, Wt, b2)


_STG = _VOCAB_PAD // 16


def _sc_pool_body(xt_hbm, t_hbm, out_hbm, idx_v, vals_v, out_v, t_sh,
                  isems, gsems):
    sub = lax.axis_index("s")
    wid = sub * 2 + lax.axis_index("c")

    def _idx_copy(l):
        return pltpu.make_async_copy(
            xt_hbm.at[l, pl.ds(wid * _RPW, _RPW)],
            idx_v.at[pl.ds(l * _RPW, _RPW)], isems.at[l // _LPC])

    def _gather_copy(c):
        return pltpu.make_async_copy(
            t_sh.at[idx_v.at[pl.ds(_CHW * c, _CHW)]],
            vals_v.at[pl.ds(_CHW * c, _CHW)], gsems.at[c])

    for l in range(_HIST):
        _idx_copy(l).start()
    pltpu.sync_copy(t_hbm.at[pl.ds(sub * _STG, _STG)],
                    t_sh.at[pl.ds(sub * _STG, _STG)])
    plsc.subcore_barrier()
    for c in range(_NCH):
        for l in range(_LPC * c, _LPC * (c + 1)):
            _idx_copy(l).wait()
        _gather_copy(c).start()
    for c in range(_NCH):
        _gather_copy(c).wait()
        for g in range(_RPW // 16):
            acc = vals_v[pl.ds(_CHW * c + 16 * g, 16)]
            for l in range(_LPC * c + 1, _LPC * (c + 1)):
                acc = acc + vals_v[pl.ds(512 * l + 16 * g, 16)]
            if c == 0:
                out_v[pl.ds(16 * g, 16)] = acc
            else:
                out_v[pl.ds(16 * g, 16)] = out_v[pl.ds(16 * g, 16)] + acc
    for g in range(_RPW // 16):
        z = out_v[pl.ds(16 * g, 16)]
        out_v[pl.ds(16 * g, 16)] = 1.0 / (1.0 + jnp.exp(-z))
    pltpu.sync_copy(out_v, out_hbm.at[pl.ds(wid * _RPW, _RPW)])


_sc_pool = functools.partial(
    pl.kernel,
    out_type=jax.ShapeDtypeStruct((_BATCH,), jnp.float32),
    mesh=plsc.VectorSubcoreMesh(core_axis_name="c", subcore_axis_name="s"),
    scratch_types=[
        pltpu.VMEM((_IPW,), jnp.int32),
        pltpu.VMEM((_IPW,), jnp.float32),
        pltpu.VMEM((_RPW,), jnp.float32),
        pltpu.VMEM_SHARED((_VOCAB_PAD,), jnp.float32),
        pltpu.SemaphoreType.DMA((_NCH,)),
        pltpu.SemaphoreType.DMA((_NCH,)),
    ],
)(_sc_pool_body)


def kernel(x, emb_table, W, b):
    t2 = _build_table(emb_table.T, W.reshape(1, _DIM), b.reshape(1, 1))
    y = _sc_pool(x.T, t2.reshape(-1))
    return y.reshape(_BATCH, 1)
